# Initial kernel scaffold; baseline (speedup 1.0000x reference)
import jax
import jax.numpy as jnp
from jax.experimental import pallas as pl


def kernel(samples, context, w_table, b_table):
    raise NotImplementedError("write your pallas kernel here")


# speedup vs baseline: 6.4158x; 6.4158x over previous
import jax
import jax.numpy as jnp
from jax import lax
from jax.experimental import pallas as pl
from jax.experimental.pallas import tpu as pltpu
from jax.experimental.pallas import tpu_sc as plsc

B = 16384
D = 128
S = 17
L = 16

NC = 2
NS = 16
NW = NC * NS

ROWS_W = B // NW
ROWS_C = 4
PAIRS_C = ROWS_C * S
PAD_C = 72
CHUNKS_W = ROWS_W // ROWS_C
NSEG = D // L


def _sc_body(idx_hbm, ctx_hbm, w_hbm, b_hbm, out_hbm,
             idx_v, ctx_v, wbuf0, wbuf1, bias_v, logits_v,
             semw0, semw1, semb0, semb1):
    wid = lax.axis_index("s") * NC + lax.axis_index("c")
    row0 = wid * ROWS_W
    c0 = wid * CHUNKS_W

    pltpu.sync_copy(idx_hbm.at[pl.ds(c0, CHUNKS_W)], idx_v)
    pltpu.sync_copy(ctx_hbm.at[pl.ds(row0, ROWS_W)], ctx_v)

    def fire(c, wbuf, semw, semb):
        pltpu.async_copy(w_hbm.at[idx_v.at[c]], wbuf, semw)
        pltpu.async_copy(b_hbm.at[idx_v.at[c]],
                         bias_v.at[pl.ds(c * PAD_C, PAD_C)], semb)

    def drain(wbuf, semw, semb):
        pltpu.make_async_copy(w_hbm.at[pl.ds(0, PAD_C)], wbuf, semw).wait()
        pltpu.make_async_copy(b_hbm.at[pl.ds(0, PAD_C)],
                              bias_v.at[pl.ds(0, PAD_C)], semb).wait()

    lane15 = lax.iota(jnp.int32, L) == (L - 1)

    def compute(c, wbuf):
        for i in range(ROWS_C):
            b_loc = c * ROWS_C + i
            ctx = [ctx_v[b_loc, pl.ds(k * L, L)] for k in range(NSEG)]

            def s_body(s, _, i=i, ctx=ctx, b_loc=b_loc):
                row = i * S + s
                acc = wbuf[row, pl.ds(0, L)] * ctx[0]
                for k in range(1, NSEG):
                    acc = acc + wbuf[row, pl.ds(k * L, L)] * ctx[k]
                plsc.store_scatter(
                    logits_v,
                    [jnp.full((L,), b_loc, jnp.int32),
                     jnp.full((L,), s, jnp.int32)],
                    jnp.cumsum(acc), mask=lane15)
                return 0

            lax.fori_loop(0, S, s_body, 0)

    fire(0, wbuf0, semw0, semb0)
    fire(1, wbuf1, semw1, semb1)

    def loop_body(g, _):
        drain(wbuf0, semw0, semb0)
        compute(2 * g, wbuf0)

        @pl.when(g < CHUNKS_W // 2 - 1)
        def _():
            fire(2 * g + 2, wbuf0, semw0, semb0)

        drain(wbuf1, semw1, semb1)
        compute(2 * g + 1, wbuf1)

        @pl.when(g < CHUNKS_W // 2 - 1)
        def _():
            fire(2 * g + 3, wbuf1, semw1, semb1)

        return 0

    lax.fori_loop(0, CHUNKS_W // 2, loop_body, 0)

    iota = lax.iota(jnp.int32, L)

    def jblock(j, _):
        b = j * L + iota
        base = (b >> 2) * PAD_C + (b & 3) * S
        xs = []
        for s in range(S):
            lg = plsc.load_gather(logits_v, [b, jnp.full((L,), s, jnp.int32)])
            bi = plsc.load_gather(bias_v, [base + s])
            xs.append(lg + bi)
        m = xs[0]
        for s in range(1, S):
            m = jnp.maximum(m, xs[s])
        es = [jnp.exp(x - m) for x in xs]
        tot = es[0]
        for s in range(1, S):
            tot = tot + es[s]
        r = 1.0 / tot
        for s in range(S):
            plsc.store_scatter(logits_v, [b, jnp.full((L,), s, jnp.int32)],
                               es[s] * r)
        return 0

    lax.fori_loop(0, ROWS_W // L, jblock, 0)

    pltpu.sync_copy(logits_v, out_hbm.at[pl.ds(row0, ROWS_W)])


@jax.jit
def _run(idx, context, w_table, b_flat):
    mesh = plsc.VectorSubcoreMesh(core_axis_name="c", subcore_axis_name="s",
                                  num_cores=NC, num_subcores=NS)
    return pl.kernel(
        _sc_body,
        out_type=jax.ShapeDtypeStruct((B, S), jnp.float32),
        mesh=mesh,
        compiler_params=pltpu.CompilerParams(needs_layout_passes=False,
                                             use_tc_tiling_on_sc=False),
        scratch_types=[
            pltpu.VMEM((CHUNKS_W, PAD_C), jnp.int32),
            pltpu.VMEM((ROWS_W, D), jnp.float32),
            pltpu.VMEM((PAD_C, D), jnp.float32),
            pltpu.VMEM((PAD_C, D), jnp.float32),
            pltpu.VMEM((CHUNKS_W * PAD_C,), jnp.float32),
            pltpu.VMEM((ROWS_W, S), jnp.float32),
            pltpu.SemaphoreType.DMA,
            pltpu.SemaphoreType.DMA,
            pltpu.SemaphoreType.DMA,
            pltpu.SemaphoreType.DMA,
        ],
    )(idx, context, w_table, b_flat)


def kernel(samples, context, w_table, b_table):
    idx = samples.astype(jnp.int32).reshape(B * S // PAIRS_C, PAIRS_C)
    idx = jnp.pad(idx, ((0, 0), (0, PAD_C - PAIRS_C)))
    return _run(idx, context, w_table, b_table.reshape(-1))
